# baseline (speedup 1.0000x reference)
import jax
import jax.numpy as jnp
from jax import lax
from jax.experimental import pallas as pl
from jax.experimental.pallas import tpu as pltpu
from jax.experimental.pallas import tpu_sc as plsc

VOCAB = 1000000
DIM = 64
BASE = 10000.0
BATCH = 1024
SEQ = 200
LANES = 16
NQ = DIM // LANES

NUM_CORES = 2
NUM_SUBCORES = 16
NW = NUM_CORES * NUM_SUBCORES
BB = 128
NUNITS = SEQ * (BATCH // BB)
UNITS_PER_W = NUNITS // NW
TOK_PER_W = UNITS_PER_W * BB
OUT_ELEMS = BATCH * SEQ * DIM
TRS_STRIDE = 129


def _pe_block(seq_len):
    theta_ids = jnp.arange(0, DIM, 2)
    theta = 1.0 / (BASE ** (theta_ids.astype(jnp.float32) / DIM))
    pe = jnp.zeros((DIM,), dtype=jnp.float32)
    pe = pe.at[theta_ids].set(theta)
    pe = pe.at[theta_ids + 1].set(theta)
    position_ids = jnp.arange(0, seq_len).astype(jnp.float32)
    out = jnp.outer(position_ids, pe)
    return jnp.sin(out)


def _sc_body(ids2_hbm, coff_hbm, table_hbm, pe_hbm, out_hbm,
             idx_v, off_v, pe_v, rows0, rows1, trs0, trs1,
             sem_g0, sem_g1, sem_o0, sem_o1):
    wid = lax.axis_index("s") * NUM_CORES + lax.axis_index("c")
    u0 = wid * UNITS_PER_W

    pltpu.sync_copy(ids2_hbm.at[pl.ds(wid * TOK_PER_W, TOK_PER_W)], idx_v)
    pltpu.sync_copy(coff_hbm.at[pl.ds(wid * TOK_PER_W, TOK_PER_W)], off_v)
    pltpu.sync_copy(pe_hbm, pe_v)

    lane = lax.iota(jnp.int32, 16)
    qlanes = [q * LANES + lane for q in range(NQ)]

    def start_gather(i, rows, sem):
        return pltpu.async_copy(
            table_hbm.at[idx_v.at[pl.ds(i * BB, BB)]], rows, sem)

    def wait_gather(i, rows, sem):
        pltpu.make_async_copy(
            table_hbm.at[idx_v.at[pl.ds(i * BB, BB)]], rows, sem).wait()

    def compute_unit(i, rows, trs):
        u = u0 + i
        s = u // 8
        pe_regs = [pe_v[pl.ds(s * DIM + q * LANES, LANES)] for q in range(NQ)]

        def tokgrp(g, _):
            coffs = off_v[pl.ds(i * BB + g * LANES, LANES)]
            for j in range(LANES):
                t = g * LANES + j
                c0 = coffs[j]
                tvec = jnp.full((LANES,), t, jnp.int32)
                for q in range(NQ):
                    v = rows[t, pl.ds(c0 + q * LANES, LANES)] + pe_regs[q]
                    plsc.store_scatter(trs, [qlanes[q], tvec], v)
            return _

        lax.fori_loop(0, BB // LANES, tokgrp, None)

    def unit_out_slices(i):
        u = u0 + i
        s = u // 8
        blk = u - 8 * s
        return s, blk

    def fire_out(i, trs, sem):
        s, blk = unit_out_slices(i)
        for dd in range(8):
            pltpu.async_copy(
                trs.at[pl.ds(8 * dd, 8), pl.ds(0, BB)],
                out_hbm.at[s, dd, blk], sem)

    def drain_out(i, trs, sem):
        s, blk = unit_out_slices(i)
        for dd in range(8):
            pltpu.make_async_copy(
                trs.at[pl.ds(8 * dd, 8), pl.ds(0, BB)],
                out_hbm.at[s, dd, blk], sem).wait()

    start_gather(0, rows0, sem_g0)

    def pair(j, _):
        i_even = 2 * j
        i_odd = 2 * j + 1

        wait_gather(i_even, rows0, sem_g0)
        start_gather(i_odd, rows1, sem_g1)

        @pl.when(j >= 1)
        def _():
            drain_out(i_even - 2, trs0, sem_o0)

        compute_unit(i_even, rows0, trs0)
        fire_out(i_even, trs0, sem_o0)

        wait_gather(i_odd, rows1, sem_g1)

        @pl.when(j < UNITS_PER_W // 2 - 1)
        def _():
            start_gather(i_odd + 1, rows0, sem_g0)

        @pl.when(j >= 1)
        def _():
            drain_out(i_odd - 2, trs1, sem_o1)

        compute_unit(i_odd, rows1, trs1)
        fire_out(i_odd, trs1, sem_o1)
        return _

    lax.fori_loop(0, UNITS_PER_W // 2, pair, None)
    drain_out(UNITS_PER_W - 2, trs0, sem_o0)
    drain_out(UNITS_PER_W - 1, trs1, sem_o1)


@jax.jit
def _run(ids2, coff, table2, pe):
    mesh = plsc.VectorSubcoreMesh(core_axis_name="c", subcore_axis_name="s")
    f = pl.kernel(
        _sc_body,
        out_type=jax.ShapeDtypeStruct((SEQ, 8, BATCH // BB, 8, BB),
                                      jnp.float32),
        mesh=mesh,
        scratch_types=[
            pltpu.VMEM((TOK_PER_W,), jnp.int32),
            pltpu.VMEM((TOK_PER_W,), jnp.int32),
            pltpu.VMEM((SEQ * DIM,), jnp.float32),
            pltpu.VMEM((BB, BB), jnp.float32),
            pltpu.VMEM((BB, BB), jnp.float32),
            pltpu.VMEM((DIM, TRS_STRIDE), jnp.float32),
            pltpu.VMEM((DIM, TRS_STRIDE), jnp.float32),
            pltpu.SemaphoreType.DMA,
            pltpu.SemaphoreType.DMA,
            pltpu.SemaphoreType.DMA,
            pltpu.SemaphoreType.DMA,
        ],
        compiler_params=pltpu.CompilerParams(
            use_tc_tiling_on_sc=True, needs_layout_passes=False),
    )
    return f(ids2, coff, table2, pe)


def kernel(input_ids, emb_table):
    ids = input_ids.transpose(1, 0).reshape(-1).astype(jnp.int32)
    table2 = emb_table.reshape(VOCAB // 2, 2 * DIM)
    ids2 = ids // 2
    coff = (ids % 2) * DIM
    pe = _pe_block(SEQ).reshape(-1)
    out5 = _run(ids2, coff, table2, pe)
    return out5.transpose(2, 4, 0, 1, 3).reshape(BATCH, SEQ, DIM)


# speedup vs baseline: 1.2110x; 1.2110x over previous
import jax
import jax.numpy as jnp
from jax import lax
from jax.experimental import pallas as pl
from jax.experimental.pallas import tpu as pltpu
from jax.experimental.pallas import tpu_sc as plsc

VOCAB = 1000000
DIM = 64
BASE = 10000.0
BATCH = 1024
SEQ = 200
LANES = 16
NQ = DIM // LANES

NUM_CORES = 2
NUM_SUBCORES = 16
NW = NUM_CORES * NUM_SUBCORES
BB = 128
NUNITS = SEQ * (BATCH // BB)
UNITS_PER_W = NUNITS // NW
TOK_PER_W = UNITS_PER_W * BB
OUT_ELEMS = BATCH * SEQ * DIM
TRS_STRIDE = 129


def _pe_block(seq_len):
    theta_ids = jnp.arange(0, DIM, 2)
    theta = 1.0 / (BASE ** (theta_ids.astype(jnp.float32) / DIM))
    pe = jnp.zeros((DIM,), dtype=jnp.float32)
    pe = pe.at[theta_ids].set(theta)
    pe = pe.at[theta_ids + 1].set(theta)
    position_ids = jnp.arange(0, seq_len).astype(jnp.float32)
    out = jnp.outer(position_ids, pe)
    return jnp.sin(out)


def _sc_body(ids2_hbm, coff_hbm, table_hbm, pe_hbm, out_hbm,
             idx_v, off_v, pe_v, rows0, rows1, trs0, trs1,
             sem_g0, sem_g1, sem_o0, sem_o1):
    wid = lax.axis_index("s") * NUM_CORES + lax.axis_index("c")
    u0 = wid * UNITS_PER_W

    pltpu.sync_copy(ids2_hbm.at[pl.ds(wid * TOK_PER_W, TOK_PER_W)], idx_v)
    pltpu.sync_copy(coff_hbm.at[pl.ds(wid * TOK_PER_W, TOK_PER_W)], off_v)
    pltpu.sync_copy(pe_hbm, pe_v)

    lane = lax.iota(jnp.int32, 16)
    qlanes = [q * LANES + lane for q in range(NQ)]

    def start_gather(i, rows, sem):
        return pltpu.async_copy(
            table_hbm.at[idx_v.at[pl.ds(i * BB, BB)]], rows, sem)

    def wait_gather(i, rows, sem):
        pltpu.make_async_copy(
            table_hbm.at[idx_v.at[pl.ds(i * BB, BB)]], rows, sem).wait()

    def compute_unit(i, rows, trs):
        u = u0 + i
        s = u // 8
        pe_regs = [pe_v[pl.ds(s * DIM + q * LANES, LANES)] for q in range(NQ)]

        def tokgrp(g, _):
            coffs = off_v[pl.ds(i * BB + g * LANES, LANES)]
            for j in range(LANES):
                t = g * LANES + j
                c0 = coffs[j]
                tvec = jnp.full((LANES,), t, jnp.int32)
                for q in range(NQ):
                    v = rows[t, pl.ds(c0 + q * LANES, LANES)] + pe_regs[q]
                    plsc.store_scatter(trs, [qlanes[q], tvec], v)
            return _

        lax.fori_loop(0, BB // LANES, tokgrp, None)

    def unit_out_slices(i):
        u = u0 + i
        s = u // 8
        blk = u - 8 * s
        return s, blk

    def fire_out(i, trs, sem):
        s, blk = unit_out_slices(i)
        for dd in range(8):
            pltpu.async_copy(
                trs.at[pl.ds(8 * dd, 8), pl.ds(0, BB)],
                out_hbm.at[s, dd, blk], sem)

    def drain_out(i, trs, sem):
        s, blk = unit_out_slices(i)
        for dd in range(8):
            pltpu.make_async_copy(
                trs.at[pl.ds(8 * dd, 8), pl.ds(0, BB)],
                out_hbm.at[s, dd, blk], sem).wait()

    start_gather(0, rows0, sem_g0)

    def pair(j, _):
        i_even = 2 * j
        i_odd = 2 * j + 1

        wait_gather(i_even, rows0, sem_g0)
        start_gather(i_odd, rows1, sem_g1)

        @pl.when(j >= 1)
        def _():
            drain_out(i_even - 2, trs0, sem_o0)

        compute_unit(i_even, rows0, trs0)
        fire_out(i_even, trs0, sem_o0)

        wait_gather(i_odd, rows1, sem_g1)

        @pl.when(j < UNITS_PER_W // 2 - 1)
        def _():
            start_gather(i_odd + 1, rows0, sem_g0)

        @pl.when(j >= 1)
        def _():
            drain_out(i_odd - 2, trs1, sem_o1)

        compute_unit(i_odd, rows1, trs1)
        fire_out(i_odd, trs1, sem_o1)
        return _

    lax.fori_loop(0, UNITS_PER_W // 2, pair, None)
    drain_out(UNITS_PER_W - 2, trs0, sem_o0)
    drain_out(UNITS_PER_W - 1, trs1, sem_o1)


@jax.jit
def _run(ids2, coff, table2, pe):
    mesh = plsc.VectorSubcoreMesh(core_axis_name="c", subcore_axis_name="s")
    f = pl.kernel(
        _sc_body,
        out_type=jax.ShapeDtypeStruct((SEQ, 8, BATCH // BB, 8, BB),
                                      jnp.float32),
        mesh=mesh,
        scratch_types=[
            pltpu.VMEM((TOK_PER_W,), jnp.int32),
            pltpu.VMEM((TOK_PER_W,), jnp.int32),
            pltpu.VMEM((SEQ * DIM,), jnp.float32),
            pltpu.VMEM((BB, BB), jnp.float32),
            pltpu.VMEM((BB, BB), jnp.float32),
            pltpu.VMEM((DIM, TRS_STRIDE), jnp.float32),
            pltpu.VMEM((DIM, TRS_STRIDE), jnp.float32),
            pltpu.SemaphoreType.DMA,
            pltpu.SemaphoreType.DMA,
            pltpu.SemaphoreType.DMA,
            pltpu.SemaphoreType.DMA,
        ],
        compiler_params=pltpu.CompilerParams(
            use_tc_tiling_on_sc=False, needs_layout_passes=False),
    )
    return f(ids2, coff, table2, pe)


def kernel(input_ids, emb_table):
    ids = input_ids.transpose(1, 0).reshape(-1).astype(jnp.int32)
    table2 = emb_table.reshape(VOCAB // 2, 2 * DIM)
    ids2 = ids // 2
    coff = (ids % 2) * DIM
    pe = _pe_block(SEQ).reshape(-1)
    out5 = _run(ids2, coff, table2, pe)
    return out5.transpose(2, 4, 0, 1, 3).reshape(BATCH, SEQ, DIM)
